# baseline (speedup 1.0000x reference)
import functools

import jax
import jax.numpy as jnp
from jax import lax
from jax.experimental import pallas as pl
from jax.experimental.pallas import tpu as pltpu
from jax.experimental.pallas import tpu_sc as plsc

N = 10000
E = 160000
D = 256
H = 256
O = 64
HALF = H // 2

NC, NS = 2, 16
CHUNK = 80
NCHUNK = 128
NPG, PGC = 16, 8
EPAD = NS * NCHUNK * CHUNK
NPAD = 10240
ROWS_PER_TILE = NPAD // NS
DEG_ROWS = NPAD // NS

BM = 512
GRID = (N + BM - 1) // BM
EPS = 1e-5

_sc_mesh = plsc.VectorSubcoreMesh(
    core_axis_name="c", subcore_axis_name="s", num_cores=NC, num_subcores=NS
)


def _deg_body(dst_hbm, deg_hbm, dst_v, ones_v, zeros_v, acc):
    c = lax.axis_index("c")
    s = lax.axis_index("s")

    def fill(i, _):
        ones_v[pl.ds(i * 16, 16)] = jnp.ones((16,), jnp.float32)
        return 0

    lax.fori_loop(0, CHUNK // 16, fill, 0)

    def zfill(i, _):
        zeros_v[pl.ds(i * 16, 16)] = jnp.zeros((16,), jnp.float32)
        return 0

    lax.fori_loop(0, DEG_ROWS // 16, zfill, 0)

    pltpu.sync_copy(zeros_v, acc.at[pl.ds(s * DEG_ROWS, DEG_ROWS)])
    pltpu.sync_copy(dst_hbm.at[s], dst_v)
    plsc.subcore_barrier()

    def step(j, carry):
        pltpu.sync_copy(ones_v, acc.at[dst_v.at[j]], add=True)
        return carry

    lax.fori_loop(0, NCHUNK, step, 0)
    plsc.subcore_barrier()

    @pl.when(c == 0)
    def _():
        pltpu.sync_copy(
            acc.at[pl.ds(s * DEG_ROWS, DEG_ROWS)],
            deg_hbm.at[pl.ds(s * DEG_ROWS, DEG_ROWS)],
        )


_deg_call = pl.kernel(
    _deg_body,
    out_type=jax.ShapeDtypeStruct((NPAD,), jnp.float32),
    mesh=_sc_mesh,
    scratch_types=[
        pltpu.VMEM((NCHUNK, CHUNK), jnp.int32),
        pltpu.VMEM((CHUNK,), jnp.float32),
        pltpu.VMEM((DEG_ROWS,), jnp.float32),
        pltpu.VMEM_SHARED((NPAD,), jnp.float32),
    ],
)


def _prop_body(z_hbm, src_hbm, dst_hbm, y_hbm, sb0, sb1, db0, db1,
               gb0, gb1, gb2, gb3, spsem0, spsem1, dpsem0, dpsem1,
               gsem0, gsem1, gsem2, gsem3, ssem0, ssem1, ssem2, ssem3, acc):
    c = lax.axis_index("c")
    s = lax.axis_index("s")
    row0 = s * ROWS_PER_TILE
    sbs, spsems = (sb0, sb1), (spsem0, spsem1)
    dbs, dpsems = (db0, db1), (dpsem0, dpsem1)
    gbs = (gb0, gb1, gb2, gb3)
    gsems = (gsem0, gsem1, gsem2, gsem3)
    ssems = (ssem0, ssem1, ssem2, ssem3)

    pltpu.async_copy(src_hbm.at[s, 0], sb0, spsem0)
    pltpu.async_copy(dst_hbm.at[s, 0], db0, dpsem0)
    pltpu.sync_copy(
        z_hbm.at[c, pl.ds(row0, ROWS_PER_TILE)],
        acc.at[pl.ds(row0, ROWS_PER_TILE)],
    )
    plsc.subcore_barrier()

    def gather(k):
        b, p, kk = k % 4, k // PGC, k % PGC
        pltpu.async_copy(
            z_hbm.at[c].at[sbs[p % 2].at[kk]], gbs[b], gsems[b],
        )

    def wait_gather(k):
        b, p, kk = k % 4, k // PGC, k % PGC
        pltpu.make_async_copy(
            z_hbm.at[c].at[sbs[p % 2].at[kk]], gbs[b], gsems[b],
        ).wait()

    def scatter(k):
        b, p, kk = k % 4, k // PGC, k % PGC
        pltpu.async_copy(gbs[b], acc.at[dbs[p % 2].at[kk]], ssems[b], add=True)

    def wait_scatter(k):
        b, p, kk = k % 4, k // PGC, k % PGC
        pltpu.make_async_copy(
            gbs[b], acc.at[dbs[p % 2].at[kk]], ssems[b]
        ).wait()

    def wait_spage(p):
        pltpu.make_async_copy(src_hbm.at[s, p], sbs[p % 2],
                              spsems[p % 2]).wait()

    wait_spage(0)
    gather(0)
    gather(1)
    for k in range(NCHUNK):
        p, kk = k // PGC, k % PGC
        wait_gather(k)
        if kk == 0:
            pltpu.make_async_copy(dst_hbm.at[s, p], dbs[p % 2],
                                  dpsems[p % 2]).wait()
        scatter(k)
        if k >= 2:
            wait_scatter(k - 2)
        if kk == 1 and p + 1 < NPG:
            pltpu.async_copy(src_hbm.at[s, p + 1], sbs[(p + 1) % 2],
                             spsems[(p + 1) % 2])
            pltpu.async_copy(dst_hbm.at[s, p + 1], dbs[(p + 1) % 2],
                             dpsems[(p + 1) % 2])
        if k + 2 < NCHUNK:
            q = (k + 2) // PGC
            if (k + 2) % PGC == 0:
                wait_spage(q)
            gather(k + 2)
    wait_scatter(NCHUNK - 2)
    wait_scatter(NCHUNK - 1)

    plsc.subcore_barrier()
    pltpu.sync_copy(
        acc.at[pl.ds(row0, ROWS_PER_TILE)],
        y_hbm.at[c, pl.ds(row0, ROWS_PER_TILE)],
    )


_prop_call = pl.kernel(
    _prop_body,
    out_type=jax.ShapeDtypeStruct((NC, NPAD, HALF), jnp.float32),
    mesh=_sc_mesh,
    scratch_types=(
        [pltpu.VMEM((PGC, CHUNK), jnp.int32) for _ in range(4)]
        + [pltpu.VMEM((CHUNK, HALF), jnp.float32) for _ in range(4)]
        + [pltpu.SemaphoreType.DMA for _ in range(12)]
        + [pltpu.VMEM_SHARED((NPAD, HALF), jnp.float32)]
    ),
)


def _mm1_body(x_ref, deg_ref, w_ref, z_ref):
    i = pl.program_id(0)
    dinv = lax.rsqrt(deg_ref[...] + 1.0)
    z = jnp.dot(x_ref[...], w_ref[...], preferred_element_type=jnp.float32)
    z = z * dinv
    rows = i * BM + lax.broadcasted_iota(jnp.int32, (BM, 1), 0)
    z = jnp.where(rows < N, z, 0.0)
    z_ref[0] = z[:, :HALF]
    z_ref[1] = z[:, HALF:]


_mm1 = pl.pallas_call(
    _mm1_body,
    grid=(GRID,),
    in_specs=[
        pl.BlockSpec((BM, D), lambda i: (i, 0)),
        pl.BlockSpec((BM, 1), lambda i: (i, 0)),
        pl.BlockSpec((D, H), lambda i: (0, 0)),
    ],
    out_specs=pl.BlockSpec((NC, BM, HALF), lambda i: (0, i, 0)),
    out_shape=jax.ShapeDtypeStruct((NC, NPAD, HALF), jnp.float32),
)


def _bn_relu(t, st, g_ref, be_ref):
    m = st[0] * (1.0 / N)
    v = st[1] * (1.0 / N) - m * m
    a = (t - m) * lax.rsqrt(v + EPS) * g_ref[...] + be_ref[...]
    return jnp.maximum(a, 0.0)


def _stats_mm_body(y_ref, deg_ref, b_ref, g_ref, be_ref, w_ref, z_ref, st_acc):
    ph = pl.program_id(0)
    i = pl.program_id(1)
    dinv = lax.rsqrt(deg_ref[...] + 1.0)
    h = jnp.concatenate([y_ref[0], y_ref[1]], axis=1)
    t = h * dinv + b_ref[...]
    rows = i * BM + lax.broadcasted_iota(jnp.int32, (BM, 1), 0)

    @pl.when(ph == 0)
    def _():
        @pl.when(i == 0)
        def _():
            st_acc[...] = jnp.zeros_like(st_acc)

        tm = jnp.where(rows < N, t, 0.0)
        st_acc[0] += jnp.sum(tm, axis=0)
        st_acc[1] += jnp.sum(tm * tm, axis=0)

    @pl.when(ph == 1)
    def _():
        a = _bn_relu(t, st_acc, g_ref, be_ref)
        z = jnp.dot(a, w_ref[...], preferred_element_type=jnp.float32) * dinv
        z = jnp.where(rows < N, z, 0.0)
        z_ref[0] = z[:, :HALF]
        z_ref[1] = z[:, HALF:]


_stats_mm = pl.pallas_call(
    _stats_mm_body,
    grid=(2, GRID),
    in_specs=[
        pl.BlockSpec((NC, BM, HALF), lambda ph, i: (0, i, 0)),
        pl.BlockSpec((BM, 1), lambda ph, i: (i, 0)),
        pl.BlockSpec((1, H), lambda ph, i: (0, 0)),
        pl.BlockSpec((1, H), lambda ph, i: (0, 0)),
        pl.BlockSpec((1, H), lambda ph, i: (0, 0)),
        pl.BlockSpec((H, H), lambda ph, i: (0, 0)),
    ],
    out_specs=pl.BlockSpec((NC, BM, HALF), lambda ph, i: (0, i, 0)),
    out_shape=jax.ShapeDtypeStruct((NC, NPAD, HALF), jnp.float32),
    scratch_shapes=[pltpu.VMEM((2, H), jnp.float32)],
)


def _stats_head_body(y_ref, deg_ref, b_ref, g_ref, be_ref, fcw_ref, fcb_ref,
                     ow_ref, ob_ref, o_ref, st_acc):
    ph = pl.program_id(0)
    i = pl.program_id(1)
    dinv = lax.rsqrt(deg_ref[...] + 1.0)
    h = jnp.concatenate([y_ref[0], y_ref[1]], axis=1)
    t = h * dinv + b_ref[...]
    rows = i * BM + lax.broadcasted_iota(jnp.int32, (BM, 1), 0)

    @pl.when(ph == 0)
    def _():
        @pl.when(i == 0)
        def _():
            st_acc[...] = jnp.zeros_like(st_acc)

        tm = jnp.where(rows < N, t, 0.0)
        st_acc[0] += jnp.sum(tm, axis=0)
        st_acc[1] += jnp.sum(tm * tm, axis=0)

    @pl.when(ph == 1)
    def _():
        a = _bn_relu(t, st_acc, g_ref, be_ref)
        f = jnp.dot(a, fcw_ref[...], preferred_element_type=jnp.float32)
        f = jnp.maximum(f + fcb_ref[...], 0.0)
        logits = jnp.dot(f, ow_ref[...], preferred_element_type=jnp.float32)
        logits = logits + ob_ref[...]
        mx = jnp.max(logits, axis=1, keepdims=True)
        lse = mx + jnp.log(jnp.sum(jnp.exp(logits - mx), axis=1, keepdims=True))
        o_ref[...] = logits - lse


_stats_head = pl.pallas_call(
    _stats_head_body,
    grid=(2, GRID),
    in_specs=[
        pl.BlockSpec((NC, BM, HALF), lambda ph, i: (0, i, 0)),
        pl.BlockSpec((BM, 1), lambda ph, i: (i, 0)),
        pl.BlockSpec((1, H), lambda ph, i: (0, 0)),
        pl.BlockSpec((1, H), lambda ph, i: (0, 0)),
        pl.BlockSpec((1, H), lambda ph, i: (0, 0)),
        pl.BlockSpec((H, H // 2), lambda ph, i: (0, 0)),
        pl.BlockSpec((1, H // 2), lambda ph, i: (0, 0)),
        pl.BlockSpec((H // 2, O), lambda ph, i: (0, 0)),
        pl.BlockSpec((1, O), lambda ph, i: (0, 0)),
    ],
    out_specs=pl.BlockSpec((BM, O), lambda ph, i: (i, 0)),
    out_shape=jax.ShapeDtypeStruct((N, O), jnp.float32),
    scratch_shapes=[pltpu.VMEM((2, H), jnp.float32)],
)


def kernel(x, edge_index, W1, b1, g1, be1, W2, b2, g2, be2, W3, b3, g3, be3,
           fcW, fcb, outW, outb):
    pad = N + jnp.arange(EPAD - E, dtype=jnp.int32) % (NPAD - N)
    src_p = jnp.concatenate([edge_index[0], pad]).reshape(NS, NPG, PGC, CHUNK)
    dst_p = jnp.concatenate([edge_index[1], pad]).reshape(NS, NCHUNK, CHUNK)
    dst_pg = dst_p.reshape(NS, NPG, PGC, CHUNK)
    deg = _deg_call(dst_p).reshape(NPAD, 1)

    b1r, g1r, be1r = b1.reshape(1, -1), g1.reshape(1, -1), be1.reshape(1, -1)
    b2r, g2r, be2r = b2.reshape(1, -1), g2.reshape(1, -1), be2.reshape(1, -1)
    b3r, g3r, be3r = b3.reshape(1, -1), g3.reshape(1, -1), be3.reshape(1, -1)

    z1 = _mm1(x, deg, W1)
    y1 = _prop_call(z1, src_p, dst_pg)
    z2 = _stats_mm(y1, deg, b1r, g1r, be1r, W2)
    y2 = _prop_call(z2, src_p, dst_pg)
    z3 = _stats_mm(y2, deg, b2r, g2r, be2r, W3)
    y3 = _prop_call(z3, src_p, dst_pg)
    return _stats_head(y3, deg, b3r, g3r, be3r, fcW, fcb.reshape(1, -1),
                       outW, outb.reshape(1, -1))


# speedup vs baseline: 1.1067x; 1.1067x over previous
import functools

import jax
import jax.numpy as jnp
from jax import lax
from jax.experimental import pallas as pl
from jax.experimental.pallas import tpu as pltpu
from jax.experimental.pallas import tpu_sc as plsc

N = 10000
E = 160000
D = 256
H = 256
O = 64
HALF = H // 2

NC, NS = 2, 16
CHUNK = 80
NCHUNK = 128
NPG, PGC = 8, 16
EPAD = NS * NCHUNK * CHUNK
NPAD = 10240
ROWS_PER_TILE = NPAD // NS
DEG_ROWS = NPAD // NS

BM = 512
GRID = (N + BM - 1) // BM
EPS = 1e-5

_sc_mesh = plsc.VectorSubcoreMesh(
    core_axis_name="c", subcore_axis_name="s", num_cores=NC, num_subcores=NS
)


def _deg_body(dst_hbm, deg_hbm, dst_v, ones_v, zeros_v, acc):
    c = lax.axis_index("c")
    s = lax.axis_index("s")

    def fill(i, _):
        ones_v[pl.ds(i * 16, 16)] = jnp.ones((16,), jnp.float32)
        return 0

    lax.fori_loop(0, CHUNK // 16, fill, 0)

    def zfill(i, _):
        zeros_v[pl.ds(i * 16, 16)] = jnp.zeros((16,), jnp.float32)
        return 0

    lax.fori_loop(0, DEG_ROWS // 16, zfill, 0)

    pltpu.sync_copy(zeros_v, acc.at[pl.ds(s * DEG_ROWS, DEG_ROWS)])
    pltpu.sync_copy(dst_hbm.at[s], dst_v)
    plsc.subcore_barrier()

    def step(j, carry):
        pltpu.sync_copy(ones_v, acc.at[dst_v.at[j]], add=True)
        return carry

    lax.fori_loop(0, NCHUNK, step, 0)
    plsc.subcore_barrier()

    @pl.when(c == 0)
    def _():
        pltpu.sync_copy(
            acc.at[pl.ds(s * DEG_ROWS, DEG_ROWS)],
            deg_hbm.at[pl.ds(s * DEG_ROWS, DEG_ROWS)],
        )


_deg_call = pl.kernel(
    _deg_body,
    out_type=jax.ShapeDtypeStruct((NPAD,), jnp.float32),
    mesh=_sc_mesh,
    scratch_types=[
        pltpu.VMEM((NCHUNK, CHUNK), jnp.int32),
        pltpu.VMEM((CHUNK,), jnp.float32),
        pltpu.VMEM((DEG_ROWS,), jnp.float32),
        pltpu.VMEM_SHARED((NPAD,), jnp.float32),
    ],
)


def _prop_body(z_hbm, src_hbm, dst_hbm, y_hbm, src_v, pb0, pb1,
               gb0, gb1, gb2, psem0, psem1, gsem0, gsem1, gsem2,
               ssem0, ssem1, ssem2, acc):
    c = lax.axis_index("c")
    s = lax.axis_index("s")
    row0 = s * ROWS_PER_TILE
    pbs, psems = (pb0, pb1), (psem0, psem1)
    gbs, gsems, ssems = (gb0, gb1, gb2), (gsem0, gsem1, gsem2), (ssem0, ssem1, ssem2)

    pltpu.sync_copy(src_hbm.at[s], src_v)
    pltpu.async_copy(dst_hbm.at[s, 0], pb0, psem0)
    pltpu.sync_copy(
        z_hbm.at[c, pl.ds(row0, ROWS_PER_TILE)],
        acc.at[pl.ds(row0, ROWS_PER_TILE)],
    )
    plsc.subcore_barrier()

    def gather(k):
        b = k % 3
        pltpu.async_copy(
            z_hbm.at[c].at[src_v.at[pl.ds(k * CHUNK, CHUNK)]],
            gbs[b], gsems[b],
        )

    def wait_gather(k):
        b = k % 3
        pltpu.make_async_copy(
            z_hbm.at[c].at[src_v.at[pl.ds(k * CHUNK, CHUNK)]],
            gbs[b], gsems[b],
        ).wait()

    def scatter(k):
        b, p, kk = k % 3, k // PGC, k % PGC
        pltpu.async_copy(gbs[b], acc.at[pbs[p % 2].at[kk]], ssems[b], add=True)

    def wait_scatter(k):
        b, p, kk = k % 3, k // PGC, k % PGC
        pltpu.make_async_copy(
            gbs[b], acc.at[pbs[p % 2].at[kk]], ssems[b]
        ).wait()

    gather(0)
    gather(1)
    for k in range(NCHUNK):
        p, kk = k // PGC, k % PGC
        wait_gather(k)
        if kk == 0:
            pltpu.make_async_copy(dst_hbm.at[s, p], pbs[p % 2],
                                  psems[p % 2]).wait()
        scatter(k)
        if k >= 1:
            wait_scatter(k - 1)
        if kk == 0 and p + 1 < NPG:
            pltpu.async_copy(dst_hbm.at[s, p + 1], pbs[(p + 1) % 2],
                             psems[(p + 1) % 2])
        if k + 2 < NCHUNK:
            gather(k + 2)
    wait_scatter(NCHUNK - 1)

    plsc.subcore_barrier()
    pltpu.sync_copy(
        acc.at[pl.ds(row0, ROWS_PER_TILE)],
        y_hbm.at[c, pl.ds(row0, ROWS_PER_TILE)],
    )


_prop_call = pl.kernel(
    _prop_body,
    out_type=jax.ShapeDtypeStruct((NC, NPAD, HALF), jnp.float32),
    mesh=_sc_mesh,
    scratch_types=(
        [pltpu.VMEM((NCHUNK * CHUNK,), jnp.int32)]
        + [pltpu.VMEM((PGC, CHUNK), jnp.int32) for _ in range(2)]
        + [pltpu.VMEM((CHUNK, HALF), jnp.float32) for _ in range(3)]
        + [pltpu.SemaphoreType.DMA for _ in range(8)]
        + [pltpu.VMEM_SHARED((NPAD, HALF), jnp.float32)]
    ),
)


def _mm1_body(x_ref, deg_ref, w_ref, z_ref):
    i = pl.program_id(0)
    dinv = lax.rsqrt(deg_ref[...] + 1.0)
    z = jnp.dot(x_ref[...], w_ref[...], preferred_element_type=jnp.float32)
    z = z * dinv
    rows = i * BM + lax.broadcasted_iota(jnp.int32, (BM, 1), 0)
    z = jnp.where(rows < N, z, 0.0)
    z_ref[0] = z[:, :HALF]
    z_ref[1] = z[:, HALF:]


_mm1 = pl.pallas_call(
    _mm1_body,
    grid=(GRID,),
    in_specs=[
        pl.BlockSpec((BM, D), lambda i: (i, 0)),
        pl.BlockSpec((BM, 1), lambda i: (i, 0)),
        pl.BlockSpec((D, H), lambda i: (0, 0)),
    ],
    out_specs=pl.BlockSpec((NC, BM, HALF), lambda i: (0, i, 0)),
    out_shape=jax.ShapeDtypeStruct((NC, NPAD, HALF), jnp.float32),
)


def _stats_body(y_ref, deg_ref, b_ref, st_ref):
    i = pl.program_id(0)

    @pl.when(i == 0)
    def _():
        st_ref[...] = jnp.zeros_like(st_ref)

    dinv = lax.rsqrt(deg_ref[...] + 1.0)
    h = jnp.concatenate([y_ref[0], y_ref[1]], axis=1)
    t = h * dinv + b_ref[...]
    rows = i * BM + lax.broadcasted_iota(jnp.int32, (BM, 1), 0)
    t = jnp.where(rows < N, t, 0.0)
    st_ref[0] += jnp.sum(t, axis=0)
    st_ref[1] += jnp.sum(t * t, axis=0)


_stats = pl.pallas_call(
    _stats_body,
    grid=(GRID,),
    in_specs=[
        pl.BlockSpec((NC, BM, HALF), lambda i: (0, i, 0)),
        pl.BlockSpec((BM, 1), lambda i: (i, 0)),
        pl.BlockSpec((1, H), lambda i: (0, 0)),
    ],
    out_specs=pl.BlockSpec((2, H), lambda i: (0, 0)),
    out_shape=jax.ShapeDtypeStruct((2, H), jnp.float32),
)


def _bn_relu(t, st_ref, g_ref, be_ref):
    m = st_ref[0] * (1.0 / N)
    v = st_ref[1] * (1.0 / N) - m * m
    a = (t - m) * lax.rsqrt(v + EPS) * g_ref[...] + be_ref[...]
    return jnp.maximum(a, 0.0)


def _mm_mid_body(y_ref, deg_ref, b_ref, g_ref, be_ref, st_ref, w_ref, z_ref):
    i = pl.program_id(0)
    dinv = lax.rsqrt(deg_ref[...] + 1.0)
    h = jnp.concatenate([y_ref[0], y_ref[1]], axis=1)
    t = h * dinv + b_ref[...]
    a = _bn_relu(t, st_ref, g_ref, be_ref)
    z = jnp.dot(a, w_ref[...], preferred_element_type=jnp.float32) * dinv
    rows = i * BM + lax.broadcasted_iota(jnp.int32, (BM, 1), 0)
    z = jnp.where(rows < N, z, 0.0)
    z_ref[0] = z[:, :HALF]
    z_ref[1] = z[:, HALF:]


_mm_mid = pl.pallas_call(
    _mm_mid_body,
    grid=(GRID,),
    in_specs=[
        pl.BlockSpec((NC, BM, HALF), lambda i: (0, i, 0)),
        pl.BlockSpec((BM, 1), lambda i: (i, 0)),
        pl.BlockSpec((1, H), lambda i: (0, 0)),
        pl.BlockSpec((1, H), lambda i: (0, 0)),
        pl.BlockSpec((1, H), lambda i: (0, 0)),
        pl.BlockSpec((2, H), lambda i: (0, 0)),
        pl.BlockSpec((H, H), lambda i: (0, 0)),
    ],
    out_specs=pl.BlockSpec((NC, BM, HALF), lambda i: (0, i, 0)),
    out_shape=jax.ShapeDtypeStruct((NC, NPAD, HALF), jnp.float32),
)


def _head_body(y_ref, deg_ref, b_ref, g_ref, be_ref, st_ref, fcw_ref, fcb_ref,
               ow_ref, ob_ref, o_ref):
    dinv = lax.rsqrt(deg_ref[...] + 1.0)
    h = jnp.concatenate([y_ref[0], y_ref[1]], axis=1)
    t = h * dinv + b_ref[...]
    a = _bn_relu(t, st_ref, g_ref, be_ref)
    f = jnp.dot(a, fcw_ref[...], preferred_element_type=jnp.float32) + fcb_ref[...]
    f = jnp.maximum(f, 0.0)
    logits = jnp.dot(f, ow_ref[...], preferred_element_type=jnp.float32) + ob_ref[...]
    mx = jnp.max(logits, axis=1, keepdims=True)
    lse = mx + jnp.log(jnp.sum(jnp.exp(logits - mx), axis=1, keepdims=True))
    o_ref[...] = logits - lse


_head = pl.pallas_call(
    _head_body,
    grid=(GRID,),
    in_specs=[
        pl.BlockSpec((NC, BM, HALF), lambda i: (0, i, 0)),
        pl.BlockSpec((BM, 1), lambda i: (i, 0)),
        pl.BlockSpec((1, H), lambda i: (0, 0)),
        pl.BlockSpec((1, H), lambda i: (0, 0)),
        pl.BlockSpec((1, H), lambda i: (0, 0)),
        pl.BlockSpec((2, H), lambda i: (0, 0)),
        pl.BlockSpec((H, H // 2), lambda i: (0, 0)),
        pl.BlockSpec((1, H // 2), lambda i: (0, 0)),
        pl.BlockSpec((H // 2, O), lambda i: (0, 0)),
        pl.BlockSpec((1, O), lambda i: (0, 0)),
    ],
    out_specs=pl.BlockSpec((BM, O), lambda i: (i, 0)),
    out_shape=jax.ShapeDtypeStruct((N, O), jnp.float32),
)


def kernel(x, edge_index, W1, b1, g1, be1, W2, b2, g2, be2, W3, b3, g3, be3,
           fcW, fcb, outW, outb):
    pad = N + jnp.arange(EPAD - E, dtype=jnp.int32) % (NPAD - N)
    src_p = jnp.concatenate([edge_index[0], pad]).reshape(NS, NCHUNK * CHUNK)
    dst_p = jnp.concatenate([edge_index[1], pad]).reshape(NS, NCHUNK, CHUNK)
    dst_pg = dst_p.reshape(NS, NPG, PGC, CHUNK)
    deg = _deg_call(dst_p).reshape(NPAD, 1)

    b1r, g1r, be1r = b1.reshape(1, -1), g1.reshape(1, -1), be1.reshape(1, -1)
    b2r, g2r, be2r = b2.reshape(1, -1), g2.reshape(1, -1), be2.reshape(1, -1)
    b3r, g3r, be3r = b3.reshape(1, -1), g3.reshape(1, -1), be3.reshape(1, -1)

    z1 = _mm1(x, deg, W1)
    y1 = _prop_call(z1, src_p, dst_pg)
    st1 = _stats(y1, deg, b1r)
    z2 = _mm_mid(y1, deg, b1r, g1r, be1r, st1, W2)
    y2 = _prop_call(z2, src_p, dst_pg)
    st2 = _stats(y2, deg, b2r)
    z3 = _mm_mid(y2, deg, b2r, g2r, be2r, st2, W3)
    y3 = _prop_call(z3, src_p, dst_pg)
    st3 = _stats(y3, deg, b3r)
    return _head(y3, deg, b3r, g3r, be3r, st3, fcW, fcb.reshape(1, -1),
                 outW, outb.reshape(1, -1))


# speedup vs baseline: 1.1301x; 1.0211x over previous
import functools

import jax
import jax.numpy as jnp
from jax import lax
from jax.experimental import pallas as pl
from jax.experimental.pallas import tpu as pltpu
from jax.experimental.pallas import tpu_sc as plsc

N = 10000
E = 160000
D = 256
H = 256
O = 64
HALF = H // 2

NC, NS = 2, 16
CHUNK = 88
NCHUNK = 114
NPG, PGC = 19, 6
EPAD = NS * NCHUNK * CHUNK
NPAD = 10240
ROWS_PER_TILE = NPAD // NS
DEG_ROWS = NPAD // NS

BM = 512
GRID = (N + BM - 1) // BM
EPS = 1e-5

_sc_mesh = plsc.VectorSubcoreMesh(
    core_axis_name="c", subcore_axis_name="s", num_cores=NC, num_subcores=NS
)


def _deg_body(dst_hbm, deg_hbm, dst_v, ones_v, zeros_v, acc):
    c = lax.axis_index("c")
    s = lax.axis_index("s")

    def fill(i, _):
        ones_v[pl.ds(i * 16, 16)] = jnp.ones((16,), jnp.float32)
        return 0

    lax.fori_loop(0, CHUNK // 16, fill, 0)

    def zfill(i, _):
        zeros_v[pl.ds(i * 16, 16)] = jnp.zeros((16,), jnp.float32)
        return 0

    lax.fori_loop(0, DEG_ROWS // 16, zfill, 0)

    pltpu.sync_copy(zeros_v, acc.at[pl.ds(s * DEG_ROWS, DEG_ROWS)])
    pltpu.sync_copy(dst_hbm.at[s], dst_v)
    plsc.subcore_barrier()

    def step(j, carry):
        pltpu.sync_copy(ones_v, acc.at[dst_v.at[j]], add=True)
        return carry

    lax.fori_loop(0, NCHUNK, step, 0)
    plsc.subcore_barrier()

    @pl.when(c == 0)
    def _():
        pltpu.sync_copy(
            acc.at[pl.ds(s * DEG_ROWS, DEG_ROWS)],
            deg_hbm.at[pl.ds(s * DEG_ROWS, DEG_ROWS)],
        )


_deg_call = pl.kernel(
    _deg_body,
    out_type=jax.ShapeDtypeStruct((NPAD,), jnp.float32),
    mesh=_sc_mesh,
    scratch_types=[
        pltpu.VMEM((NCHUNK, CHUNK), jnp.int32),
        pltpu.VMEM((CHUNK,), jnp.float32),
        pltpu.VMEM((DEG_ROWS,), jnp.float32),
        pltpu.VMEM_SHARED((NPAD,), jnp.float32),
    ],
)


def _prop_body(z_hbm, src_hbm, dst_hbm, y_hbm, src_v, pb0, pb1,
               gb0, gb1, gb2, psem0, psem1, gsem0, gsem1, gsem2,
               ssem0, ssem1, ssem2, acc):
    c = lax.axis_index("c")
    s = lax.axis_index("s")
    row0 = s * ROWS_PER_TILE
    pbs, psems = (pb0, pb1), (psem0, psem1)
    gbs, gsems, ssems = (gb0, gb1, gb2), (gsem0, gsem1, gsem2), (ssem0, ssem1, ssem2)

    pltpu.sync_copy(src_hbm.at[s], src_v)
    pltpu.async_copy(dst_hbm.at[s, 0], pb0, psem0)
    pltpu.sync_copy(
        z_hbm.at[c, pl.ds(row0, ROWS_PER_TILE)],
        acc.at[pl.ds(row0, ROWS_PER_TILE)],
    )
    plsc.subcore_barrier()

    def gather(k):
        b = k % 3
        pltpu.async_copy(
            z_hbm.at[c].at[src_v.at[pl.ds(k * CHUNK, CHUNK)]],
            gbs[b], gsems[b],
        )

    def wait_gather(k):
        b = k % 3
        pltpu.make_async_copy(
            z_hbm.at[c].at[src_v.at[pl.ds(k * CHUNK, CHUNK)]],
            gbs[b], gsems[b],
        ).wait()

    def scatter(k):
        b, p, kk = k % 3, k // PGC, k % PGC
        pltpu.async_copy(gbs[b], acc.at[pbs[p % 2].at[kk]], ssems[b], add=True)

    def wait_scatter(k):
        b, p, kk = k % 3, k // PGC, k % PGC
        pltpu.make_async_copy(
            gbs[b], acc.at[pbs[p % 2].at[kk]], ssems[b]
        ).wait()

    gather(0)
    gather(1)
    for k in range(NCHUNK):
        p, kk = k // PGC, k % PGC
        wait_gather(k)
        if kk == 0:
            pltpu.make_async_copy(dst_hbm.at[s, p], pbs[p % 2],
                                  psems[p % 2]).wait()
        scatter(k)
        if k >= 1:
            wait_scatter(k - 1)
        if kk == 0 and p + 1 < NPG:
            pltpu.async_copy(dst_hbm.at[s, p + 1], pbs[(p + 1) % 2],
                             psems[(p + 1) % 2])
        if k + 2 < NCHUNK:
            gather(k + 2)
    wait_scatter(NCHUNK - 1)

    plsc.subcore_barrier()
    pltpu.sync_copy(
        acc.at[pl.ds(row0, ROWS_PER_TILE)],
        y_hbm.at[c, pl.ds(row0, ROWS_PER_TILE)],
    )


_prop_call = pl.kernel(
    _prop_body,
    out_type=jax.ShapeDtypeStruct((NC, NPAD, HALF), jnp.float32),
    mesh=_sc_mesh,
    scratch_types=(
        [pltpu.VMEM((NCHUNK * CHUNK,), jnp.int32)]
        + [pltpu.VMEM((PGC, CHUNK), jnp.int32) for _ in range(2)]
        + [pltpu.VMEM((CHUNK, HALF), jnp.float32) for _ in range(3)]
        + [pltpu.SemaphoreType.DMA for _ in range(8)]
        + [pltpu.VMEM_SHARED((NPAD, HALF), jnp.float32)]
    ),
)


def _mm1_body(x_ref, deg_ref, w_ref, z_ref):
    i = pl.program_id(0)
    dinv = lax.rsqrt(deg_ref[...] + 1.0)
    z = jnp.dot(x_ref[...], w_ref[...], preferred_element_type=jnp.float32)
    z = z * dinv
    rows = i * BM + lax.broadcasted_iota(jnp.int32, (BM, 1), 0)
    z = jnp.where(rows < N, z, 0.0)
    z_ref[0] = z[:, :HALF]
    z_ref[1] = z[:, HALF:]


_mm1 = pl.pallas_call(
    _mm1_body,
    grid=(GRID,),
    in_specs=[
        pl.BlockSpec((BM, D), lambda i: (i, 0)),
        pl.BlockSpec((BM, 1), lambda i: (i, 0)),
        pl.BlockSpec((D, H), lambda i: (0, 0)),
    ],
    out_specs=pl.BlockSpec((NC, BM, HALF), lambda i: (0, i, 0)),
    out_shape=jax.ShapeDtypeStruct((NC, NPAD, HALF), jnp.float32),
)


def _stats_body(y_ref, deg_ref, b_ref, st_ref):
    i = pl.program_id(0)

    @pl.when(i == 0)
    def _():
        st_ref[...] = jnp.zeros_like(st_ref)

    dinv = lax.rsqrt(deg_ref[...] + 1.0)
    h = jnp.concatenate([y_ref[0], y_ref[1]], axis=1)
    t = h * dinv + b_ref[...]
    rows = i * BM + lax.broadcasted_iota(jnp.int32, (BM, 1), 0)
    t = jnp.where(rows < N, t, 0.0)
    st_ref[0] += jnp.sum(t, axis=0)
    st_ref[1] += jnp.sum(t * t, axis=0)


_stats = pl.pallas_call(
    _stats_body,
    grid=(GRID,),
    in_specs=[
        pl.BlockSpec((NC, BM, HALF), lambda i: (0, i, 0)),
        pl.BlockSpec((BM, 1), lambda i: (i, 0)),
        pl.BlockSpec((1, H), lambda i: (0, 0)),
    ],
    out_specs=pl.BlockSpec((2, H), lambda i: (0, 0)),
    out_shape=jax.ShapeDtypeStruct((2, H), jnp.float32),
)


def _bn_relu(t, st_ref, g_ref, be_ref):
    m = st_ref[0] * (1.0 / N)
    v = st_ref[1] * (1.0 / N) - m * m
    a = (t - m) * lax.rsqrt(v + EPS) * g_ref[...] + be_ref[...]
    return jnp.maximum(a, 0.0)


def _mm_mid_body(y_ref, deg_ref, b_ref, g_ref, be_ref, st_ref, w_ref, z_ref):
    i = pl.program_id(0)
    dinv = lax.rsqrt(deg_ref[...] + 1.0)
    h = jnp.concatenate([y_ref[0], y_ref[1]], axis=1)
    t = h * dinv + b_ref[...]
    a = _bn_relu(t, st_ref, g_ref, be_ref)
    z = jnp.dot(a, w_ref[...], preferred_element_type=jnp.float32) * dinv
    rows = i * BM + lax.broadcasted_iota(jnp.int32, (BM, 1), 0)
    z = jnp.where(rows < N, z, 0.0)
    z_ref[0] = z[:, :HALF]
    z_ref[1] = z[:, HALF:]


_mm_mid = pl.pallas_call(
    _mm_mid_body,
    grid=(GRID,),
    in_specs=[
        pl.BlockSpec((NC, BM, HALF), lambda i: (0, i, 0)),
        pl.BlockSpec((BM, 1), lambda i: (i, 0)),
        pl.BlockSpec((1, H), lambda i: (0, 0)),
        pl.BlockSpec((1, H), lambda i: (0, 0)),
        pl.BlockSpec((1, H), lambda i: (0, 0)),
        pl.BlockSpec((2, H), lambda i: (0, 0)),
        pl.BlockSpec((H, H), lambda i: (0, 0)),
    ],
    out_specs=pl.BlockSpec((NC, BM, HALF), lambda i: (0, i, 0)),
    out_shape=jax.ShapeDtypeStruct((NC, NPAD, HALF), jnp.float32),
)


def _head_body(y_ref, deg_ref, b_ref, g_ref, be_ref, st_ref, fcw_ref, fcb_ref,
               ow_ref, ob_ref, o_ref):
    dinv = lax.rsqrt(deg_ref[...] + 1.0)
    h = jnp.concatenate([y_ref[0], y_ref[1]], axis=1)
    t = h * dinv + b_ref[...]
    a = _bn_relu(t, st_ref, g_ref, be_ref)
    f = jnp.dot(a, fcw_ref[...], preferred_element_type=jnp.float32) + fcb_ref[...]
    f = jnp.maximum(f, 0.0)
    logits = jnp.dot(f, ow_ref[...], preferred_element_type=jnp.float32) + ob_ref[...]
    mx = jnp.max(logits, axis=1, keepdims=True)
    lse = mx + jnp.log(jnp.sum(jnp.exp(logits - mx), axis=1, keepdims=True))
    o_ref[...] = logits - lse


_head = pl.pallas_call(
    _head_body,
    grid=(GRID,),
    in_specs=[
        pl.BlockSpec((NC, BM, HALF), lambda i: (0, i, 0)),
        pl.BlockSpec((BM, 1), lambda i: (i, 0)),
        pl.BlockSpec((1, H), lambda i: (0, 0)),
        pl.BlockSpec((1, H), lambda i: (0, 0)),
        pl.BlockSpec((1, H), lambda i: (0, 0)),
        pl.BlockSpec((2, H), lambda i: (0, 0)),
        pl.BlockSpec((H, H // 2), lambda i: (0, 0)),
        pl.BlockSpec((1, H // 2), lambda i: (0, 0)),
        pl.BlockSpec((H // 2, O), lambda i: (0, 0)),
        pl.BlockSpec((1, O), lambda i: (0, 0)),
    ],
    out_specs=pl.BlockSpec((BM, O), lambda i: (i, 0)),
    out_shape=jax.ShapeDtypeStruct((N, O), jnp.float32),
)


def kernel(x, edge_index, W1, b1, g1, be1, W2, b2, g2, be2, W3, b3, g3, be3,
           fcW, fcb, outW, outb):
    pad = N + jnp.arange(EPAD - E, dtype=jnp.int32) % (NPAD - N)
    src_p = jnp.concatenate([edge_index[0], pad]).reshape(NS, NCHUNK * CHUNK)
    dst_p = jnp.concatenate([edge_index[1], pad]).reshape(NS, NCHUNK, CHUNK)
    dst_pg = dst_p.reshape(NS, NPG, PGC, CHUNK)
    deg = _deg_call(dst_p).reshape(NPAD, 1)

    b1r, g1r, be1r = b1.reshape(1, -1), g1.reshape(1, -1), be1.reshape(1, -1)
    b2r, g2r, be2r = b2.reshape(1, -1), g2.reshape(1, -1), be2.reshape(1, -1)
    b3r, g3r, be3r = b3.reshape(1, -1), g3.reshape(1, -1), be3.reshape(1, -1)

    z1 = _mm1(x, deg, W1)
    y1 = _prop_call(z1, src_p, dst_pg)
    st1 = _stats(y1, deg, b1r)
    z2 = _mm_mid(y1, deg, b1r, g1r, be1r, st1, W2)
    y2 = _prop_call(z2, src_p, dst_pg)
    st2 = _stats(y2, deg, b2r)
    z3 = _mm_mid(y2, deg, b2r, g2r, be2r, st2, W3)
    y3 = _prop_call(z3, src_p, dst_pg)
    st3 = _stats(y3, deg, b3r)
    return _head(y3, deg, b3r, g3r, be3r, st3, fcW, fcb.reshape(1, -1),
                 outW, outb.reshape(1, -1))
